# Initial kernel scaffold; baseline (speedup 1.0000x reference)
import jax
import jax.numpy as jnp
from jax.experimental import pallas as pl


def kernel(x, edge_index, W1, b1, W2, b2):
    raise NotImplementedError("write your pallas kernel here")


# speedup vs baseline: 27.8363x; 27.8363x over previous
import dataclasses
import functools

import jax
import jax.numpy as jnp
from jax import lax
from jax.experimental import pallas as pl
from jax.experimental.pallas import tpu as pltpu
from jax.experimental.pallas import tpu_sc as plsc

N = 10000
E = 320000
C_IN = 128
C_HID = 32

NC = 2
NS = 16
L = 16

CHUNK = 128
NCHUNK = 79
EDGES_PER_WORKER = CHUNK * NCHUNK
E_PAD = EDGES_PER_WORKER * NC * NS
N_PAD = 10112
ROWS_PER_SUB = N_PAD // NS

_mesh = plsc.VectorSubcoreMesh(core_axis_name="c", subcore_axis_name="s")

_sc_params = pltpu.CompilerParams()
if "needs_layout_passes" in pltpu.CompilerParams.__dataclass_fields__:
    _sc_params = dataclasses.replace(_sc_params, needs_layout_passes=False)
if "use_tc_tiling_on_sc" in pltpu.CompilerParams.__dataclass_fields__:
    _sc_params = dataclasses.replace(_sc_params, use_tc_tiling_on_sc=False)


@functools.partial(
    pl.kernel,
    out_type=jax.ShapeDtypeStruct((NC * NS, N_PAD), jnp.float32),
    mesh=_mesh,
    compiler_params=_sc_params,
    scratch_types=[
        pltpu.VMEM((NCHUNK, CHUNK), jnp.int32),
        pltpu.VMEM((N_PAD,), jnp.float32),
    ],
)
def _sc_deg(dst_hbm, out_hbm, dst_v, hist_v):
    cid = lax.axis_index("c")
    sid = lax.axis_index("s")
    wid = cid * NS + sid
    pltpu.sync_copy(dst_hbm.at[cid, sid], dst_v)

    @pl.loop(0, N_PAD, step=L)
    def _zero(i):
        hist_v[pl.ds(i, L)] = jnp.zeros((L,), jnp.float32)

    ones = jnp.ones((L,), jnp.float32)

    @pl.loop(0, NCHUNK)
    def _chunk(j):
        @pl.loop(0, CHUNK, step=L)
        def _vec(k):
            idx = dst_v[j, pl.ds(k, L)]
            plsc.addupdate_scatter(hist_v, [idx], ones)

    pltpu.sync_copy(hist_v, out_hbm.at[wid])


@functools.partial(
    pl.kernel,
    out_type=jax.ShapeDtypeStruct((NC, N_PAD, C_HID), jnp.float32),
    mesh=_mesh,
    compiler_params=_sc_params,
    scratch_types=[
        pltpu.VMEM((NCHUNK, CHUNK), jnp.int32),
        pltpu.VMEM((NCHUNK, CHUNK), jnp.int32),
        pltpu.VMEM((CHUNK, C_HID), jnp.float32),
        pltpu.VMEM_SHARED((N_PAD, C_HID), jnp.float32),
    ],
)
def _sc_agg(table_hbm, src_hbm, dst_hbm, zeros_hbm, out_hbm,
            src_v, dst_v, rows_v, acc_sh):
    cid = lax.axis_index("c")
    sid = lax.axis_index("s")
    pltpu.sync_copy(src_hbm.at[cid, sid], src_v)
    pltpu.sync_copy(dst_hbm.at[cid, sid], dst_v)
    base = sid * ROWS_PER_SUB
    pltpu.sync_copy(zeros_hbm.at[pl.ds(base, ROWS_PER_SUB)],
                    acc_sh.at[pl.ds(base, ROWS_PER_SUB)])
    plsc.subcore_barrier()

    @pl.loop(0, NCHUNK)
    def _chunk(j):
        pltpu.sync_copy(table_hbm.at[src_v.at[j]], rows_v)
        pltpu.sync_copy(rows_v, acc_sh.at[dst_v.at[j]], add=True)

    plsc.subcore_barrier()
    pltpu.sync_copy(acc_sh.at[pl.ds(base, ROWS_PER_SUB)],
                    out_hbm.at[cid, pl.ds(base, ROWS_PER_SUB)])


def _mm1_body(x_ref, w_ref, o_ref):
    o_ref[...] = jnp.dot(x_ref[...], w_ref[...],
                         preferred_element_type=jnp.float32)


def _scale_body(parts_ref, h1_ref, dinv_ref, t1_ref):
    deg = jnp.sum(parts_ref[...], axis=0)[:N] + 1.0
    dinv = lax.rsqrt(deg)
    dinv_ref[...] = dinv
    t1_ref[...] = h1_ref[...] * dinv[:, None]


def _gelu(v):
    return 0.5 * v * (1.0 + lax.erf(v * (2.0 ** -0.5)))


def _mid_body(agg_ref, t1_ref, dinv_ref, b1_ref, w2_ref, t2_ref):
    dinv = dinv_ref[...][:, None]
    s = agg_ref[0, :N, :] + agg_ref[1, :N, :] + t1_ref[...]
    out1 = dinv * s + b1_ref[...][None, :]
    g = _gelu(out1)
    h2 = jnp.dot(g, w2_ref[...], preferred_element_type=jnp.float32)
    t2_ref[...] = dinv * h2


def _final_body(agg_ref, t2_ref, dinv_ref, b2_ref, o_ref):
    dinv = dinv_ref[...][:, None]
    s = agg_ref[0, :N, :] + agg_ref[1, :N, :] + t2_ref[...]
    o_ref[...] = dinv * s + b2_ref[...][None, :]


def kernel(x, edge_index, W1, b1, W2, b2):
    src = edge_index[0].astype(jnp.int32)
    dst = edge_index[1].astype(jnp.int32)
    pad = E_PAD - E
    src_p = jnp.concatenate([src, jnp.zeros((pad,), jnp.int32)])
    dst_p = jnp.concatenate([dst, jnp.full((pad,), N, jnp.int32)])
    src_p = src_p.reshape(NC, NS, NCHUNK, CHUNK)
    dst_p = dst_p.reshape(NC, NS, NCHUNK, CHUNK)
    zeros = jnp.zeros((N_PAD, C_HID), jnp.float32)

    deg_parts = _sc_deg(dst_p)

    h1 = pl.pallas_call(
        _mm1_body,
        out_shape=jax.ShapeDtypeStruct((N, C_HID), jnp.float32),
    )(x, W1)

    dinv, t1 = pl.pallas_call(
        _scale_body,
        out_shape=(jax.ShapeDtypeStruct((N,), jnp.float32),
                   jax.ShapeDtypeStruct((N, C_HID), jnp.float32)),
    )(deg_parts, h1)

    agg1 = _sc_agg(t1, src_p, dst_p, zeros)

    t2 = pl.pallas_call(
        _mid_body,
        out_shape=jax.ShapeDtypeStruct((N, C_HID), jnp.float32),
    )(agg1, t1, dinv, b1, W2)

    agg2 = _sc_agg(t2, src_p, dst_p, zeros)

    out = pl.pallas_call(
        _final_body,
        out_shape=jax.ShapeDtypeStruct((N, C_HID), jnp.float32),
    )(agg2, t2, dinv, b2)

    return out
